# Initial kernel scaffold; baseline (speedup 1.0000x reference)
import jax
import jax.numpy as jnp
from jax.experimental import pallas as pl


def kernel(x, edge_index, edge_attr, W1, as1, ad1, We1, ae1, b1, W2, as2, ad2, We2, ae2, b2, W3, as3, ad3, We3, ae3, b3):
    raise NotImplementedError("write your pallas kernel here")


# speedup vs baseline: 1.0036x; 1.0036x over previous
import functools

import jax
import jax.numpy as jnp
from jax.experimental import pallas as pl
from jax.experimental.pallas import tpu as pltpu


def _bias_kernel(x_ref, b_ref, o_ref):
    o_ref[...] = x_ref[...] + b_ref[0, 0]


def _add_bias_pallas(x, b):
    n = x.shape[0]
    blk = 2000
    grid = (n // blk,)
    return pl.pallas_call(
        _bias_kernel,
        grid=grid,
        in_specs=[
            pl.BlockSpec((blk, 1), lambda i: (i, 0)),
            pl.BlockSpec(memory_space=pltpu.SMEM),
        ],
        out_specs=pl.BlockSpec((blk, 1), lambda i: (i, 0)),
        out_shape=jax.ShapeDtypeStruct((n, 1), jnp.float32),
    )(x, b.reshape(1, 1))


def _with_self_loops(edge_index, edge_attr, n):
    loop = jnp.arange(n, dtype=edge_index.dtype)
    ei = jnp.concatenate([edge_index, jnp.stack([loop, loop])], axis=1)
    mean_ea = jnp.mean(edge_attr, axis=0, keepdims=True)
    ea = jnp.concatenate([edge_attr, jnp.broadcast_to(mean_ea, (n, edge_attr.shape[1]))], axis=0)
    return ei, ea


def _gat(x, src, dst, ea, W, a_s, a_d, We, a_e, b, H, C, concat, add_bias=True):
    n = x.shape[0]
    h = (x @ W).reshape(n, H, C)
    he = (ea @ We).reshape(-1, H, C)
    alpha = (h * a_s).sum(-1)[src] + (h * a_d).sum(-1)[dst] + (he * a_e).sum(-1)
    alpha = jax.nn.leaky_relu(alpha, 0.2)
    amax = jax.ops.segment_max(alpha, dst, num_segments=n)
    amax = jnp.where(jnp.isneginf(amax), 0.0, amax)
    ex = jnp.exp(alpha - amax[dst])
    den = jax.ops.segment_sum(ex, dst, num_segments=n)
    alpha = ex / (den[dst] + 1e-16)
    out = jax.ops.segment_sum(h[src] * alpha[:, :, None], dst, num_segments=n)
    out = out.reshape(n, H * C) if concat else out.mean(axis=1)
    if add_bias:
        out = out + b
    return out


def kernel(x, edge_index, edge_attr, W1, as1, ad1, We1, ae1, b1, W2, as2, ad2, We2, ae2, b2, W3, as3, ad3, We3, ae3, b3):
    ei, ea = _with_self_loops(edge_index, edge_attr, x.shape[0])
    src, dst = ei[0], ei[1]
    h = jax.nn.elu(_gat(x, src, dst, ea, W1, as1, ad1, We1, ae1, b1, 4, 32, True))
    h = jax.nn.elu(_gat(h, src, dst, ea, W2, as2, ad2, We2, ae2, b2, 4, 32, True))
    out = _gat(h, src, dst, ea, W3, as3, ad3, We3, ae3, b3, 1, 1, False, add_bias=False)
    return _add_bias_pallas(out, b3)


# speedup vs baseline: 4.8610x; 4.8437x over previous
import jax
import jax.numpy as jnp
from jax.experimental import pallas as pl
from jax.experimental.pallas import tpu as pltpu


def _node_kernel(x_ref, w_ref, as_ref, ad_ref, h_ref, ss_ref, sd_ref):
    h = jnp.dot(x_ref[...], w_ref[...], preferred_element_type=jnp.float32)
    h_ref[...] = h
    ss_ref[...] = jnp.dot(h, as_ref[...], preferred_element_type=jnp.float32)
    sd_ref[...] = jnp.dot(h, ad_ref[...], preferred_element_type=jnp.float32)


def _node_pass(x, W, As, Ad, blk=1000):
    n, d = x.shape
    f = W.shape[1]
    H = As.shape[1]
    return pl.pallas_call(
        _node_kernel,
        grid=(n // blk,),
        in_specs=[
            pl.BlockSpec((blk, d), lambda i: (i, 0)),
            pl.BlockSpec((d, f), lambda i: (0, 0)),
            pl.BlockSpec((f, H), lambda i: (0, 0)),
            pl.BlockSpec((f, H), lambda i: (0, 0)),
        ],
        out_specs=[
            pl.BlockSpec((blk, f), lambda i: (i, 0)),
            pl.BlockSpec((blk, H), lambda i: (i, 0)),
            pl.BlockSpec((blk, H), lambda i: (i, 0)),
        ],
        out_shape=[
            jax.ShapeDtypeStruct((n, f), jnp.float32),
            jax.ShapeDtypeStruct((n, H), jnp.float32),
            jax.ShapeDtypeStruct((n, H), jnp.float32),
        ],
    )(x, W, As, Ad)


def _ex_kernel(ss_ref, sd_ref, ea_ref, m_ref, ub_ref, ex_ref):
    a = ss_ref[...] + sd_ref[...] + jnp.dot(
        ea_ref[...], m_ref[...], preferred_element_type=jnp.float32
    )
    a = jnp.where(a > 0, a, 0.2 * a)
    ex_ref[...] = jnp.exp(a - ub_ref[...])


def _ex_pass(ss_g, sd_g, ea, M, ub, blk=1000):
    e, H = ss_g.shape
    de = ea.shape[1]
    return pl.pallas_call(
        _ex_kernel,
        grid=(e // blk,),
        in_specs=[
            pl.BlockSpec((blk, H), lambda i: (i, 0)),
            pl.BlockSpec((blk, H), lambda i: (i, 0)),
            pl.BlockSpec((blk, de), lambda i: (i, 0)),
            pl.BlockSpec((de, H), lambda i: (0, 0)),
            pl.BlockSpec((1, H), lambda i: (0, 0)),
        ],
        out_specs=pl.BlockSpec((blk, H), lambda i: (i, 0)),
        out_shape=jax.ShapeDtypeStruct((e, H), jnp.float32),
    )(ss_g, sd_g, ea, M, ub)


def _msg_kernel(hs_ref, ex_ref, k_ref, msg_ref):
    msg_ref[...] = hs_ref[...] * jnp.dot(
        ex_ref[...], k_ref[...], preferred_element_type=jnp.float32
    )


def _msg_pass(h_src, ex, K, blk=1000):
    e, f = h_src.shape
    H = ex.shape[1]
    return pl.pallas_call(
        _msg_kernel,
        grid=(e // blk,),
        in_specs=[
            pl.BlockSpec((blk, f), lambda i: (i, 0)),
            pl.BlockSpec((blk, H), lambda i: (i, 0)),
            pl.BlockSpec((H, f), lambda i: (0, 0)),
        ],
        out_specs=pl.BlockSpec((blk, f), lambda i: (i, 0)),
        out_shape=jax.ShapeDtypeStruct((e, f), jnp.float32),
    )(h_src, ex, K)


def _norm_kernel_elu(num_ref, den_ref, k_ref, b_ref, o_ref):
    den = jnp.dot(den_ref[...], k_ref[...], preferred_element_type=jnp.float32)
    o = num_ref[...] / (den + 1e-16) + b_ref[...]
    o_ref[...] = jnp.where(o > 0, o, jnp.exp(o) - 1.0)


def _norm_kernel_plain(num_ref, den_ref, k_ref, b_ref, o_ref):
    den = jnp.dot(den_ref[...], k_ref[...], preferred_element_type=jnp.float32)
    o_ref[...] = num_ref[...] / (den + 1e-16) + b_ref[...]


def _norm_pass(num, den, K, b, elu, blk=1000):
    n, f = num.shape
    H = den.shape[1]
    body = _norm_kernel_elu if elu else _norm_kernel_plain
    return pl.pallas_call(
        body,
        grid=(n // blk,),
        in_specs=[
            pl.BlockSpec((blk, f), lambda i: (i, 0)),
            pl.BlockSpec((blk, H), lambda i: (i, 0)),
            pl.BlockSpec((H, f), lambda i: (0, 0)),
            pl.BlockSpec((1, f), lambda i: (0, 0)),
        ],
        out_specs=pl.BlockSpec((blk, f), lambda i: (i, 0)),
        out_shape=jax.ShapeDtypeStruct((n, f), jnp.float32),
    )(num, den, K, b)


def _amat(a):
    H, C = a.shape
    return (a[:, :, None] * jnp.eye(H, dtype=a.dtype)[:, None, :]).reshape(H * C, H)


def _kmat(H, C):
    return jnp.repeat(jnp.eye(H, dtype=jnp.float32), C, axis=1)


def _gat_layer(x, src, dst, ea, W, a_s, a_d, We, a_e, b, H, C, elu):
    n = x.shape[0]
    f = H * C
    As = _amat(a_s)
    Ad = _amat(a_d)
    M = We @ _amat(a_e)
    K = _kmat(H, C)

    h, ss, sd = _node_pass(x, W, As, Ad)

    ea_lo = jnp.min(ea)
    ea_hi = jnp.max(ea)
    se_max = jnp.maximum(M * ea_lo, M * ea_hi)
    ub_a = (
        jnp.max(ss, axis=0, keepdims=True)
        + jnp.max(sd, axis=0, keepdims=True)
        + se_max
    )
    ub = jnp.where(ub_a > 0, ub_a, 0.2 * ub_a)

    ss_g = jnp.take(ss, src, axis=0)
    sd_g = jnp.take(sd, dst, axis=0)
    ex = _ex_pass(ss_g, sd_g, ea, M, ub)

    den = jax.ops.segment_sum(ex, dst, num_segments=n)
    h_src = jnp.take(h, src, axis=0)
    msg = _msg_pass(h_src, ex, K)
    num = jax.ops.segment_sum(msg, dst, num_segments=n)

    return _norm_pass(num, den, K, b.reshape(1, f), elu)


def _with_self_loops(edge_index, edge_attr, n):
    loop = jnp.arange(n, dtype=edge_index.dtype)
    ei = jnp.concatenate([edge_index, jnp.stack([loop, loop])], axis=1)
    mean_ea = jnp.mean(edge_attr, axis=0, keepdims=True)
    ea = jnp.concatenate(
        [edge_attr, jnp.broadcast_to(mean_ea, (n, edge_attr.shape[1]))], axis=0
    )
    return ei, ea


def kernel(x, edge_index, edge_attr, W1, as1, ad1, We1, ae1, b1, W2, as2, ad2, We2, ae2, b2, W3, as3, ad3, We3, ae3, b3):
    ei, ea = _with_self_loops(edge_index, edge_attr, x.shape[0])
    src, dst = ei[0], ei[1]
    h = _gat_layer(x, src, dst, ea, W1, as1, ad1, We1, ae1, b1, 4, 32, True)
    h = _gat_layer(h, src, dst, ea, W2, as2, ad2, We2, ae2, b2, 4, 32, True)
    return _gat_layer(h, src, dst, ea, W3, as3, ad3, We3, ae3, b3, 1, 1, False)
